# Initial kernel scaffold; baseline (speedup 1.0000x reference)
import jax
import jax.numpy as jnp
from jax.experimental import pallas as pl


def kernel(x, edge_index, batch, percent, params):
    raise NotImplementedError("write your pallas kernel here")


# speedup vs baseline: 1.0003x; 1.0003x over previous
import jax
import jax.numpy as jnp
from jax.experimental import pallas as pl
from jax.experimental.pallas import tpu as pltpu
from jax.scipy.special import logsumexp

N_NODES = 10000
NGRAPH = 64
LAYERS = 3
DIM = 128
TEMP = 0.1
BN_EPS = 1e-5


def _mlp_body(x_ref, w1_ref, b1_ref, w2_ref, b2_ref, o_ref):
    a = jnp.maximum(
        jnp.dot(x_ref[...], w1_ref[...], preferred_element_type=jnp.float32)
        + b1_ref[...], 0.0)
    o_ref[...] = (
        jnp.dot(a, w2_ref[...], preferred_element_type=jnp.float32)
        + b2_ref[...])


def _ini_mlp(x, w1, b1, w2, b2):
    R = x.shape[0]
    BR = 1000
    grid = R // BR
    return pl.pallas_call(
        _mlp_body,
        grid=(grid,),
        in_specs=[
            pl.BlockSpec((BR, DIM), lambda i: (i, 0)),
            pl.BlockSpec((DIM, DIM), lambda i: (0, 0)),
            pl.BlockSpec((1, DIM), lambda i: (0, 0)),
            pl.BlockSpec((DIM, DIM), lambda i: (0, 0)),
            pl.BlockSpec((1, DIM), lambda i: (0, 0)),
        ],
        out_specs=pl.BlockSpec((BR, DIM), lambda i: (i, 0)),
        out_shape=jax.ShapeDtypeStruct((R, DIM), jnp.float32),
    )(x, w1, b1.reshape(1, DIM), w2, b2.reshape(1, DIM))


def kernel(x, edge_index, batch, percent, params):
    p = params
    h = _ini_mlp(x, p['ini_w1'], p['ini_b1'], p['ini_w2'], p['ini_b2'])

    counts = jnp.bincount(batch, length=NGRAPH)
    starts = jnp.concatenate([jnp.zeros((1,), counts.dtype), jnp.cumsum(counts)[:-1]])
    r = jax.random.uniform(jax.random.key(123), (N_NODES,))
    cb = counts[batch]
    offs = jnp.minimum(jnp.floor(r * cb.astype(jnp.float32)).astype(batch.dtype),
                       jnp.maximum(cb - 1, 0).astype(batch.dtype))
    neg = h[starts[batch].astype(batch.dtype) + offs]
    h2 = jnp.concatenate([h, neg], axis=0)
    offset = edge_index[0, -1] + 1
    ei = jnp.concatenate([edge_index, edge_index + offset], axis=1)
    src, dst = ei[0], ei[1]
    embeds = []
    for i in range(LAYERS):
        g = p['gin'][i]
        agg = jnp.zeros_like(h2).at[dst].add(h2[src])
        z = h2 + agg
        z = jnp.maximum(z @ g['w1'] + g['b1'], 0.0) @ g['w2'] + g['b2']
        z = jnp.maximum(z, 0.0)
        mu = z.mean(axis=0)
        var = z.var(axis=0)
        z = p['bn_g'][i] * (z - mu) / jnp.sqrt(var + BN_EPS) + p['bn_b'][i]
        h2 = z
        embeds.append(z)
    stacked = jnp.stack(embeds, axis=1)
    glob = jnp.einsum('nld,l->nd', stacked, p['layer_w']) + p['layer_b']
    glob = glob.reshape(2, N_NODES, DIM)
    pos, negn = glob[0], glob[1]
    neg_graph = jax.ops.segment_sum(negn, batch, num_segments=NGRAPH)
    Pm = jax.nn.softmax(jnp.einsum('nd,sdt->snt', pos, p['mha']), axis=-1)
    mask = (Pm[:, :, 0] >= 0.5).astype(pos.dtype)
    sub_embed = jnp.einsum('snd,s->nd', pos[None, :, :] * mask[:, :, None], p['sub_w']) + p['sub_b']
    sampling_subgraph = jax.ops.segment_sum(sub_embed, batch, num_segments=NGRAPH)
    prob_loss = -jnp.sum(jnp.abs(Pm[:, :, 0] - Pm[:, :, 1]))
    pos_graph = jax.ops.segment_sum(pos, batch, num_segments=NGRAPH)

    def l2n(a):
        return a / (jnp.linalg.norm(a, axis=-1, keepdims=True) + 1e-8)
    q = l2n(pos)
    pz = l2n(pos_graph[0])
    nk = l2n(pos_graph)
    logits = jnp.concatenate([(q @ pz)[:, None], q @ nk.T], axis=1) / TEMP
    loss_per = -(logits[:, 0] - logsumexp(logits, axis=1))
    per_graph = jax.ops.segment_sum(loss_per, batch, num_segments=NGRAPH) / jnp.maximum(counts, 1).astype(jnp.float32)
    info_nce = per_graph.mean()
    return (pos_graph, sampling_subgraph, neg_graph, prob_loss, info_nce)


# speedup vs baseline: 3.2507x; 3.2497x over previous
import functools

import jax
import jax.numpy as jnp
from jax import lax
from jax.experimental import pallas as pl
from jax.experimental.pallas import tpu as pltpu
from jax.experimental.pallas import tpu_sc as plsc

N_NODES = 10000
N2 = 20000
DIM = 128
HALF = 64
LAYERS = 3
NGRAPH = 64
TEMP = 0.1
BN_EPS = 1e-5

E2 = 640000
CH = 128
EPAD = 655360
CHPT = 320
ACCROWS = 20096
TRASH = 20000
GPAD = 12288
ROWS_PT = 1250

_f32 = jnp.float32


def _sc_mesh():
    return plsc.VectorSubcoreMesh(
        core_axis_name="c", subcore_axis_name="s", num_cores=2, num_subcores=16)


_SC_PARAMS = pltpu.CompilerParams(use_tc_tiling_on_sc=False)


def _scatter_body(znA, znB, srcp, dstp, zinit, aggA, aggB,
                  sidx, didx, rows, acc, sem):
    c = lax.axis_index("c")
    s = lax.axis_index("s")

    pltpu.sync_copy(zinit, acc.at[pl.ds(s * 1256, 1256)])
    plsc.subcore_barrier()

    def make_chunk(tab):
        def chunk(i, _):
            off = (s * CHPT + i) * CH
            pltpu.sync_copy(srcp.at[pl.ds(off, CH)], sidx)
            pltpu.sync_copy(dstp.at[pl.ds(off, CH)], didx.at[0])
            pltpu.async_copy(tab.at[sidx], rows, sem).wait()
            pltpu.sync_copy(rows, acc.at[didx.at[0]], add=True)
            return 0
        return chunk

    @pl.when(c == 0)
    def _():
        lax.fori_loop(0, CHPT, make_chunk(znA), 0)

    @pl.when(c == 1)
    def _():
        lax.fori_loop(0, CHPT, make_chunk(znB), 0)

    plsc.subcore_barrier()

    @pl.when(c == 0)
    def _():
        pltpu.sync_copy(acc.at[pl.ds(s * ROWS_PT, ROWS_PT)],
                        aggA.at[pl.ds(s * ROWS_PT, ROWS_PT)])

    @pl.when(c == 1)
    def _():
        pltpu.sync_copy(acc.at[pl.ds(s * ROWS_PT, ROWS_PT)],
                        aggB.at[pl.ds(s * ROWS_PT, ROWS_PT)])


def _sc_scatter(znA, znB, srcp, dstp, zinit):
    return pl.kernel(
        _scatter_body,
        out_type=(jax.ShapeDtypeStruct((N2, HALF), _f32),
                  jax.ShapeDtypeStruct((N2, HALF), _f32)),
        mesh=_sc_mesh(),
        scratch_types=[
            pltpu.VMEM((CH,), jnp.int32),
            pltpu.VMEM((1, CH), jnp.int32),
            pltpu.VMEM((CH, HALF), _f32),
            pltpu.VMEM_SHARED((ACCROWS, HALF), _f32),
            pltpu.SemaphoreType.DMA,
        ],
        compiler_params=_SC_PARAMS,
    )(znA, znB, srcp, dstp, zinit)


def _gather_body(hA, hB, gidx, negA, negB, sidx, rows, sem):
    c = lax.axis_index("c")
    s = lax.axis_index("s")

    def make_chunk(tab, out):
        def chunk(i, _):
            off = (s * 6 + i) * CH
            pltpu.sync_copy(gidx.at[pl.ds(off, CH)], sidx)
            pltpu.async_copy(tab.at[sidx], rows, sem).wait()
            pltpu.sync_copy(rows, out.at[pl.ds(off, CH)])
            return 0
        return chunk

    @pl.when(c == 0)
    def _():
        lax.fori_loop(0, 6, make_chunk(hA, negA), 0)

    @pl.when(c == 1)
    def _():
        lax.fori_loop(0, 6, make_chunk(hB, negB), 0)


def _sc_gather_neg(hA, hB, gidx):
    return pl.kernel(
        _gather_body,
        out_type=(jax.ShapeDtypeStruct((GPAD, HALF), _f32),
                  jax.ShapeDtypeStruct((GPAD, HALF), _f32)),
        mesh=_sc_mesh(),
        scratch_types=[
            pltpu.VMEM((CH,), jnp.int32),
            pltpu.VMEM((CH, HALF), _f32),
            pltpu.SemaphoreType.DMA,
        ],
        compiler_params=_SC_PARAMS,
    )(hA, hB, gidx)


def _ini_body(x_ref, w1_ref, b1_ref, w2_ref, b2_ref, a_ref, b_ref):
    a = jnp.maximum(
        jnp.dot(x_ref[...], w1_ref[...], preferred_element_type=_f32)
        + b1_ref[...], 0.0)
    h = jnp.dot(a, w2_ref[...], preferred_element_type=_f32) + b2_ref[...]
    a_ref[...] = h[:, :HALF]
    b_ref[...] = h[:, HALF:]


def _ini_mlp(x, w1, b1, w2, b2):
    BR = 2000
    return pl.pallas_call(
        _ini_body,
        grid=(N_NODES // BR,),
        in_specs=[
            pl.BlockSpec((BR, DIM), lambda i: (i, 0)),
            pl.BlockSpec((DIM, DIM), lambda i: (0, 0)),
            pl.BlockSpec((1, DIM), lambda i: (0, 0)),
            pl.BlockSpec((DIM, DIM), lambda i: (0, 0)),
            pl.BlockSpec((1, DIM), lambda i: (0, 0)),
        ],
        out_specs=[pl.BlockSpec((BR, HALF), lambda i: (i, 0)),
                   pl.BlockSpec((BR, HALF), lambda i: (i, 0))],
        out_shape=[jax.ShapeDtypeStruct((N_NODES, HALF), _f32),
                   jax.ShapeDtypeStruct((N_NODES, HALF), _f32)],
    )(x, w1, b1.reshape(1, DIM), w2, b2.reshape(1, DIM))


def _layer_body(znA, znB, agA, agB, w1, b1, w2, b2,
                z_ref, zs_ref, zq_ref):
    i = pl.program_id(0)
    vA = znA[...] + agA[...]
    vB = znB[...] + agB[...]
    v = jnp.concatenate([vA, vB], axis=1)
    a = jnp.maximum(
        jnp.dot(v, w1[...], preferred_element_type=_f32)
        + b1[...], 0.0)
    z = jnp.maximum(
        jnp.dot(a, w2[...], preferred_element_type=_f32) + b2[...], 0.0)
    z_ref[...] = z

    @pl.when(i == 0)
    def _():
        zs_ref[...] = jnp.zeros_like(zs_ref)
        zq_ref[...] = jnp.zeros_like(zq_ref)

    zs_ref[...] += jnp.sum(z, axis=0, keepdims=True)
    zq_ref[...] += jnp.sum(z * z, axis=0, keepdims=True)


def _gin_layer(znA, znB, agA, agB, w1, b1, w2, b2):
    BR = 2000
    return pl.pallas_call(
        _layer_body,
        grid=(N2 // BR,),
        in_specs=[
            pl.BlockSpec((BR, HALF), lambda i: (i, 0)),
            pl.BlockSpec((BR, HALF), lambda i: (i, 0)),
            pl.BlockSpec((BR, HALF), lambda i: (i, 0)),
            pl.BlockSpec((BR, HALF), lambda i: (i, 0)),
            pl.BlockSpec((DIM, DIM), lambda i: (0, 0)),
            pl.BlockSpec((1, DIM), lambda i: (0, 0)),
            pl.BlockSpec((DIM, DIM), lambda i: (0, 0)),
            pl.BlockSpec((1, DIM), lambda i: (0, 0)),
        ],
        out_specs=[pl.BlockSpec((BR, DIM), lambda i: (i, 0)),
                   pl.BlockSpec((1, DIM), lambda i: (0, 0)),
                   pl.BlockSpec((1, DIM), lambda i: (0, 0))],
        out_shape=[jax.ShapeDtypeStruct((N2, DIM), _f32),
                   jax.ShapeDtypeStruct((1, DIM), _f32),
                   jax.ShapeDtypeStruct((1, DIM), _f32)],
    )(znA, znB, agA, agB, w1,
      b1.reshape(1, DIM), w2, b2.reshape(1, DIM))


def _norm_body(z_ref, s_ref, t_ref, a_ref, b_ref):
    zn = z_ref[...] * s_ref[...] + t_ref[...]
    a_ref[...] = zn[:, :HALF]
    b_ref[...] = zn[:, HALF:]


def _norm_split(z, s, t):
    BR = 2000
    return pl.pallas_call(
        _norm_body,
        grid=(N2 // BR,),
        in_specs=[
            pl.BlockSpec((BR, DIM), lambda i: (i, 0)),
            pl.BlockSpec((1, DIM), lambda i: (0, 0)),
            pl.BlockSpec((1, DIM), lambda i: (0, 0)),
        ],
        out_specs=[pl.BlockSpec((BR, HALF), lambda i: (i, 0)),
                   pl.BlockSpec((BR, HALF), lambda i: (i, 0))],
        out_shape=[jax.ShapeDtypeStruct((N2, HALF), _f32),
                   jax.ShapeDtypeStruct((N2, HALF), _f32)],
    )(z, s, t)


def _comb_pos_body(z0, z1, z2, st_ref, mhad_ref, subw_ref, M_ref,
                   pos_ref, pg_ref, sub_ref, pls_ref):
    i = pl.program_id(0)
    st = st_ref[...]
    glob = (z0[...] * st[0:1] + z1[...] * st[1:2] + z2[...] * st[2:3]
            + st[3:4])
    pos_ref[...] = glob
    d = jnp.dot(glob, mhad_ref[...], preferred_element_type=_f32)
    mask = (d >= 0.0).astype(_f32)
    coef = jnp.sum(mask * subw_ref[...], axis=1, keepdims=True)
    Mb = M_ref[...]
    pg = lax.dot_general(Mb, glob, (((0,), (0,)), ((), ())),
                         preferred_element_type=_f32)
    sb = lax.dot_general(Mb, glob * coef, (((0,), (0,)), ((), ())),
                         preferred_element_type=_f32)

    @pl.when(i == 0)
    def _():
        pg_ref[...] = jnp.zeros_like(pg_ref)
        sub_ref[...] = jnp.zeros_like(sub_ref)
        pls_ref[...] = jnp.zeros_like(pls_ref)

    pg_ref[...] += pg
    sub_ref[...] += sb
    pls_ref[...] += jnp.sum(jnp.tanh(jnp.abs(d) * 0.5), axis=0, keepdims=True)


def _combine_pos(z0, z1, z2, st, mhad, subw, M):
    BR = 1000
    return pl.pallas_call(
        _comb_pos_body,
        grid=(N_NODES // BR,),
        in_specs=[
            pl.BlockSpec((BR, DIM), lambda i: (i, 0)),
            pl.BlockSpec((BR, DIM), lambda i: (i, 0)),
            pl.BlockSpec((BR, DIM), lambda i: (i, 0)),
            pl.BlockSpec((8, DIM), lambda i: (0, 0)),
            pl.BlockSpec((DIM, 8), lambda i: (0, 0)),
            pl.BlockSpec((1, 8), lambda i: (0, 0)),
            pl.BlockSpec((BR, NGRAPH), lambda i: (i, 0)),
        ],
        out_specs=[pl.BlockSpec((BR, DIM), lambda i: (i, 0)),
                   pl.BlockSpec((NGRAPH, DIM), lambda i: (0, 0)),
                   pl.BlockSpec((NGRAPH, DIM), lambda i: (0, 0)),
                   pl.BlockSpec((1, 8), lambda i: (0, 0))],
        out_shape=[jax.ShapeDtypeStruct((N_NODES, DIM), _f32),
                   jax.ShapeDtypeStruct((NGRAPH, DIM), _f32),
                   jax.ShapeDtypeStruct((NGRAPH, DIM), _f32),
                   jax.ShapeDtypeStruct((1, 8), _f32)],
    )(z0, z1, z2, st, mhad, subw, M)


def _comb_neg_body(z0, z1, z2, st_ref, M_ref, ng_ref):
    i = pl.program_id(0)
    st = st_ref[...]
    glob = (z0[...] * st[0:1] + z1[...] * st[1:2] + z2[...] * st[2:3]
            + st[3:4])
    ng = lax.dot_general(M_ref[...], glob, (((0,), (0,)), ((), ())),
                         preferred_element_type=_f32)

    @pl.when(i == 0)
    def _():
        ng_ref[...] = jnp.zeros_like(ng_ref)

    ng_ref[...] += ng


def _combine_neg(z0, z1, z2, st, M):
    BR = 1000
    nblk = N_NODES // BR
    return pl.pallas_call(
        _comb_neg_body,
        grid=(nblk,),
        in_specs=[
            pl.BlockSpec((BR, DIM), lambda i: (i + nblk, 0)),
            pl.BlockSpec((BR, DIM), lambda i: (i + nblk, 0)),
            pl.BlockSpec((BR, DIM), lambda i: (i + nblk, 0)),
            pl.BlockSpec((8, DIM), lambda i: (0, 0)),
            pl.BlockSpec((BR, NGRAPH), lambda i: (i, 0)),
        ],
        out_specs=pl.BlockSpec((NGRAPH, DIM), lambda i: (0, 0)),
        out_shape=jax.ShapeDtypeStruct((NGRAPH, DIM), _f32),
    )(z0, z1, z2, st, M)


def _nce_body(pos_ref, pgr_ref, M_ref, out_ref):
    i = pl.program_id(0)
    pg = pgr_ref[...]
    nn = jnp.sqrt(jnp.sum(pg * pg, axis=1, keepdims=True))
    nk = pg / (nn + 1e-8)
    pb = pos_ref[...]
    qn = jnp.sqrt(jnp.sum(pb * pb, axis=1, keepdims=True))
    q = pb / (qn + 1e-8)
    H = lax.dot_general(q, nk, (((1,), (1,)), ((), ())),
                        preferred_element_type=_f32) * (1.0 / TEMP)
    m = jnp.max(H, axis=1, keepdims=True)
    lse = m + jnp.log(jnp.exp(H[:, 0:1] - m)
                      + jnp.sum(jnp.exp(H - m), axis=1, keepdims=True))
    loss = lse - H[:, 0:1]
    lossb = jnp.broadcast_to(loss, (loss.shape[0], NGRAPH))
    acc = lax.dot_general(M_ref[...], lossb, (((0,), (0,)), ((), ())),
                          preferred_element_type=_f32)

    @pl.when(i == 0)
    def _():
        out_ref[...] = jnp.zeros_like(out_ref)

    out_ref[...] += acc


def _nce(pos, pos_graph, M):
    BR = 1000
    return pl.pallas_call(
        _nce_body,
        grid=(N_NODES // BR,),
        in_specs=[
            pl.BlockSpec((BR, DIM), lambda i: (i, 0)),
            pl.BlockSpec((NGRAPH, DIM), lambda i: (0, 0)),
            pl.BlockSpec((BR, NGRAPH), lambda i: (i, 0)),
        ],
        out_specs=pl.BlockSpec((NGRAPH, NGRAPH), lambda i: (0, 0)),
        out_shape=jax.ShapeDtypeStruct((NGRAPH, NGRAPH), _f32),
    )(pos, pos_graph, M)


def kernel(x, edge_index, batch, percent, params):
    p = params

    counts = jnp.bincount(batch, length=NGRAPH)
    starts = jnp.concatenate(
        [jnp.zeros((1,), counts.dtype), jnp.cumsum(counts)[:-1]])
    r = jax.random.uniform(jax.random.key(123), (N_NODES,))
    cb = counts[batch]
    offs = jnp.minimum(
        jnp.floor(r * cb.astype(_f32)).astype(batch.dtype),
        jnp.maximum(cb - 1, 0).astype(batch.dtype))
    gidx = starts[batch].astype(batch.dtype) + offs
    gidx_pad = jnp.concatenate(
        [gidx, jnp.zeros((GPAD - N_NODES,), jnp.int32)])

    offset = edge_index[0, -1] + 1
    src, dst = edge_index[0], edge_index[1]
    srcp = jnp.concatenate(
        [src, src + offset, jnp.zeros((EPAD - E2,), jnp.int32)])
    dstp = jnp.concatenate(
        [dst, dst + offset, jnp.full((EPAD - E2,), TRASH, jnp.int32)])

    zinit = jnp.zeros((1256, HALF), _f32)
    M = (batch[:, None] == jnp.arange(NGRAPH, dtype=batch.dtype)[None, :]
         ).astype(_f32)

    hA, hB = _ini_mlp(x, p['ini_w1'], p['ini_b1'], p['ini_w2'], p['ini_b2'])
    negA, negB = _sc_gather_neg(hA, hB, gidx_pad)
    curA = jnp.concatenate([hA, negA[:N_NODES]], axis=0)
    curB = jnp.concatenate([hB, negB[:N_NODES]], axis=0)

    zs_list, st_list = [], []
    for l in range(LAYERS):
        g = p['gin'][l]
        aggA, aggB = _sc_scatter(curA, curB, srcp, dstp, zinit)
        z, zsum, zsq = _gin_layer(curA, curB, aggA, aggB,
                                  g['w1'], g['b1'], g['w2'], g['b2'])
        mu = zsum[0] / N2
        var = zsq[0] / N2 - mu * mu
        s_l = p['bn_g'][l] / jnp.sqrt(var + BN_EPS)
        t_l = p['bn_b'][l] - mu * s_l
        zs_list.append(z)
        st_list.append((s_l, t_l))
        if l < LAYERS - 1:
            curA, curB = _norm_split(z, s_l.reshape(1, DIM),
                                     t_l.reshape(1, DIM))

    lw = p['layer_w']
    cs = jnp.stack([lw[l] * st_list[l][0] for l in range(LAYERS)])
    ct = (sum(lw[l] * st_list[l][1] for l in range(LAYERS))
          + p['layer_b'])
    st = jnp.concatenate([cs, ct.reshape(1, DIM),
                          jnp.zeros((4, DIM), _f32)], axis=0)

    mhad = jnp.concatenate(
        [(p['mha'][:, :, 0] - p['mha'][:, :, 1]).T,
         jnp.zeros((DIM, 8 - 4), _f32)], axis=1)
    subw = jnp.concatenate([p['sub_w'], jnp.zeros((4,), _f32)]
                           ).reshape(1, 8)

    pos, pos_graph, sub_acc, pls = _combine_pos(
        zs_list[0], zs_list[1], zs_list[2], st, mhad, subw, M)
    neg_graph = _combine_neg(zs_list[0], zs_list[1], zs_list[2], st, M)

    sampling_subgraph = sub_acc + counts[:, None].astype(_f32) * p['sub_b']
    prob_loss = -jnp.sum(pls)

    nce_acc = _nce(pos, pos_graph, M)
    per_graph = nce_acc[:, 0] / jnp.maximum(counts, 1).astype(_f32)
    info_nce = per_graph.mean()

    return (pos_graph, sampling_subgraph, neg_graph, prob_loss, info_nce)
